# Initial kernel scaffold; baseline (speedup 1.0000x reference)
import jax
import jax.numpy as jnp
from jax.experimental import pallas as pl


def kernel(x, edge_attr, org_x, org_edge_attr, lg_node_idx, edge_index, org_edge_index):
    raise NotImplementedError("write your pallas kernel here")


# speedup vs baseline: 3.0053x; 3.0053x over previous
import jax
import jax.numpy as jnp
from jax import lax
from jax.experimental import pallas as pl
from jax.experimental.pallas import tpu as pltpu
from jax.experimental.pallas import tpu_sc as plsc

NC = 2
NS = 16
L = 16


def _zf():
    return jnp.zeros((L,), jnp.float32)


def _of():
    return jnp.ones((L,), jnp.float32)


def _mesh():
    return plsc.VectorSubcoreMesh(core_axis_name="c", subcore_axis_name="s")


_NROWS = 160000
_NC_OUT = 5000
_NPAD = 5120
_NB = 128
_NCHUNKS = _NROWS // _NB
_NTILE = _NPAD // NS
_NFIN = ((0, 128), (128, 128), (256, 64))


def _node_body(x_hbm, i0_hbm, i1_hbm, out_hbm,
               acc0, acc1, cnt0, cnt1,
               xf, xb, i0v, i1v, l0v, l1v, onesv, c0v, c1v, gsem, ssem):
    c = lax.axis_index("c")
    s = lax.axis_index("s")

    def zrow(r, _):
        for j in range(8):
            xf[r, pl.ds(j * L, L)] = _zf()
        return 0
    lax.fori_loop(0, _NB, zrow, 0)
    for j in range((_NB + L) // L):
        c0v[pl.ds(j * L, L)] = _zf()
    for j in range(_NB // L):
        onesv[pl.ds(j * L, L)] = _of()

    base = c * _NC_OUT
    z0 = s * _NTILE
    for (off, n) in _NFIN:
        pltpu.sync_copy(xf.at[pl.ds(0, n)], acc0.at[pl.ds(z0 + off, n)])
        pltpu.sync_copy(xf.at[pl.ds(0, n)], acc1.at[pl.ds(z0 + off, n)])
        pltpu.sync_copy(c0v.at[pl.ds(0, n)], cnt0.at[pl.ds(z0 + off, n)])
        pltpu.sync_copy(c0v.at[pl.ds(0, n)], cnt1.at[pl.ds(z0 + off, n)])
    plsc.subcore_barrier()

    trips = jnp.where(s < _NCHUNKS % NS, _NCHUNKS // NS + 1, _NCHUNKS // NS)

    def chunk(k, _):
        r0 = (s + k * NS) * _NB
        d0 = pltpu.async_copy(x_hbm.at[pl.ds(r0, _NB), pl.ds(0, 128)], xf, gsem)
        d1 = pltpu.async_copy(x_hbm.at[pl.ds(r0, _NB), pl.ds(128, 128)], xb, gsem)
        d2 = pltpu.async_copy(i0_hbm.at[pl.ds(r0, _NB)], i0v, gsem)
        d3 = pltpu.async_copy(i1_hbm.at[pl.ds(r0, _NB)], i1v, gsem)
        d0.wait(); d1.wait(); d2.wait(); d3.wait()
        for j in range(_NB // L):
            v0 = i0v[pl.ds(j * L, L)]
            lo0 = v0 - base
            g0 = _NC_OUT + (v0 & 31)
            oob0 = (lo0 < 0) | (lo0 >= _NC_OUT)
            l0v[pl.ds(j * L, L)] = jnp.where(oob0, g0, lo0)
            v1 = i1v[pl.ds(j * L, L)]
            lo1 = v1 - base
            g1 = _NC_OUT + (v1 & 31)
            oob1 = (lo1 < 0) | (lo1 >= _NC_OUT)
            l1v[pl.ds(j * L, L)] = jnp.where(oob1, g1, lo1)
        w0 = pltpu.async_copy(xb, acc0.at[l0v], ssem, add=True)
        w1 = pltpu.async_copy(onesv, cnt0.at[l0v], ssem, add=True)
        w2 = pltpu.async_copy(xf, acc1.at[l1v], ssem, add=True)
        w3 = pltpu.async_copy(onesv, cnt1.at[l1v], ssem, add=True)
        w0.wait(); w1.wait(); w2.wait(); w3.wait()
        return 0
    lax.fori_loop(0, trips, chunk, 0)
    plsc.subcore_barrier()

    for (off, n) in _NFIN:
        lo = z0 + off
        pltpu.sync_copy(acc0.at[pl.ds(lo, n)], xf.at[pl.ds(0, n)])
        pltpu.sync_copy(acc1.at[pl.ds(lo, n)], xb.at[pl.ds(0, n)])
        pltpu.sync_copy(cnt0.at[pl.ds(lo, n)], c0v.at[pl.ds(0, n)])
        pltpu.sync_copy(cnt1.at[pl.ds(lo, n)], c1v.at[pl.ds(0, n)])

        def frow(r, _):
            s0 = c0v[pl.ds(r, L)][0]
            s1 = c1v[pl.ds(r, L)][0]
            rec0 = 1.0 / jnp.maximum(jnp.full((L,), s0, jnp.float32), 1.0)
            rec1 = 1.0 / jnp.maximum(jnp.full((L,), s1, jnp.float32), 1.0)
            for j in range(8):
                a = xf[r, pl.ds(j * L, L)]
                b = xb[r, pl.ds(j * L, L)]
                xf[r, pl.ds(j * L, L)] = a * rec0 + b * rec1
            return 0
        lax.fori_loop(0, n, frow, 0)
        pltpu.sync_copy(xf.at[pl.ds(0, n)],
                        out_hbm.at[pl.ds(c * _NPAD + lo, n)])


def _node_call(x, i0, i1):
    kfn = pl.kernel(
        _node_body,
        out_type=jax.ShapeDtypeStruct((2 * _NPAD, 128), jnp.float32),
        mesh=_mesh(),
        scratch_types=[
            pltpu.VMEM_SHARED((_NPAD, 128), jnp.float32),
            pltpu.VMEM_SHARED((_NPAD, 128), jnp.float32),
            pltpu.VMEM_SHARED((_NPAD,), jnp.float32),
            pltpu.VMEM_SHARED((_NPAD,), jnp.float32),
            pltpu.VMEM((_NB, 128), jnp.float32),
            pltpu.VMEM((_NB, 128), jnp.float32),
            pltpu.VMEM((_NB,), jnp.int32),
            pltpu.VMEM((_NB,), jnp.int32),
            pltpu.VMEM((_NB,), jnp.int32),
            pltpu.VMEM((_NB,), jnp.int32),
            pltpu.VMEM((_NB,), jnp.float32),
            pltpu.VMEM((_NB + L,), jnp.float32),
            pltpu.VMEM((_NB + L,), jnp.float32),
            pltpu.SemaphoreType.DMA,
            pltpu.SemaphoreType.DMA,
        ],
    )
    return kfn(x, i0, i1)


_EROWS = 1600000
_EC_OUT = 40000
_EPAD = 40960
_EW = _EPAD * L
_EB = 128
_ECHUNKS = _EROWS // _EB
_ETILE = _EPAD // NS
_EWT = _ETILE * L


def _edge_body(eafT_hbm, eabT_hbm, e0_hbm, e1_hbm, out_hbm,
               acc0, acc1, cnt0, cnt1,
               fT, bT, l0m, l1m, i0v, i1v, l0v, l1v,
               onesv, zb, fa, fb, ca, cb, gsem, ssem):
    c = lax.axis_index("c")
    s = lax.axis_index("s")

    def zrow(r, _):
        zb[pl.ds(r * L, L)] = _zf()
        return 0
    lax.fori_loop(0, _EB, zrow, 0)
    for j in range(_EB // L):
        onesv[pl.ds(j * L, L)] = _of()

    trips = jnp.where(s < _ECHUNKS % NS, _ECHUNKS // NS + 1, _ECHUNKS // NS)
    t0 = s * _ETILE
    w0 = s * _EWT

    for p in range(2):
        q = c * 2 + p
        base = q * _EC_OUT

        for t in range(_EWT // 2048):
            pltpu.sync_copy(zb, acc0.at[pl.ds(w0 + t * 2048, 2048)])
            pltpu.sync_copy(zb, acc1.at[pl.ds(w0 + t * 2048, 2048)])
        pltpu.sync_copy(zb, cnt0.at[pl.ds(t0, 2048)])
        pltpu.sync_copy(zb.at[pl.ds(0, _ETILE - 2048)],
                        cnt0.at[pl.ds(t0 + 2048, _ETILE - 2048)])
        pltpu.sync_copy(zb, cnt1.at[pl.ds(t0, 2048)])
        pltpu.sync_copy(zb.at[pl.ds(0, _ETILE - 2048)],
                        cnt1.at[pl.ds(t0 + 2048, _ETILE - 2048)])
        plsc.subcore_barrier()

        def chunk(k, _):
            r0 = (s + k * NS) * _EB
            d0 = pltpu.async_copy(
                eafT_hbm.at[pl.ds(0, L), pl.ds(r0, _EB)], fT, gsem)
            d1 = pltpu.async_copy(
                eabT_hbm.at[pl.ds(0, L), pl.ds(r0, _EB)], bT, gsem)
            d2 = pltpu.async_copy(e0_hbm.at[pl.ds(r0, _EB)], i0v, gsem)
            d3 = pltpu.async_copy(e1_hbm.at[pl.ds(r0, _EB)], i1v, gsem)
            d0.wait(); d1.wait(); d2.wait(); d3.wait()
            for j in range(_EB // L):
                v0 = i0v[pl.ds(j * L, L)]
                lo0 = v0 - base
                g0 = _EC_OUT + (v0 & 255)
                oob0 = (lo0 < 0) | (lo0 >= _EC_OUT)
                le0 = jnp.where(oob0, g0, lo0)
                l0v[pl.ds(j * L, L)] = le0
                b0 = le0 * L
                v1 = i1v[pl.ds(j * L, L)]
                lo1 = v1 - base
                g1 = _EC_OUT + (v1 & 255)
                oob1 = (lo1 < 0) | (lo1 >= _EC_OUT)
                le1 = jnp.where(oob1, g1, lo1)
                l1v[pl.ds(j * L, L)] = le1
                b1 = le1 * L
                for l in range(L):
                    l0m[l, pl.ds(j * L, L)] = b0 + l
                    l1m[l, pl.ds(j * L, L)] = b1 + l
            ws = []
            for l in range(L):
                ws.append(pltpu.async_copy(
                    bT.at[l], acc0.at[l0m.at[l]], ssem, add=True))
                ws.append(pltpu.async_copy(
                    fT.at[l], acc1.at[l1m.at[l]], ssem, add=True))
            ws.append(pltpu.async_copy(onesv, cnt0.at[l0v], ssem, add=True))
            ws.append(pltpu.async_copy(onesv, cnt1.at[l1v], ssem, add=True))
            for w in ws:
                w.wait()
            return 0
        lax.fori_loop(0, trips, chunk, 0)
        plsc.subcore_barrier()

        pltpu.sync_copy(cnt0.at[pl.ds(t0, _ETILE)], ca.at[pl.ds(0, _ETILE)])
        pltpu.sync_copy(cnt1.at[pl.ds(t0, _ETILE)], cb.at[pl.ds(0, _ETILE)])
        for t in range(_EWT // 2048):
            pltpu.sync_copy(acc0.at[pl.ds(w0 + t * 2048, 2048)], fa)
            pltpu.sync_copy(acc1.at[pl.ds(w0 + t * 2048, 2048)], fb)

            def frow(r, _):
                s0 = ca[pl.ds(t * _EB + r, L)][0]
                s1 = cb[pl.ds(t * _EB + r, L)][0]
                rec0 = 1.0 / jnp.maximum(jnp.full((L,), s0, jnp.float32), 1.0)
                rec1 = 1.0 / jnp.maximum(jnp.full((L,), s1, jnp.float32), 1.0)
                a = fa[pl.ds(r * L, L)]
                b = fb[pl.ds(r * L, L)]
                fa[pl.ds(r * L, L)] = a * rec0 + b * rec1
                return 0
            lax.fori_loop(0, _EB, frow, 0)
            pltpu.sync_copy(fa, out_hbm.at[pl.ds(q * _EW + w0 + t * 2048,
                                                 2048)])
        plsc.subcore_barrier()


def _edge_call(eafT, eabT, e0, e1):
    kfn = pl.kernel(
        _edge_body,
        out_type=jax.ShapeDtypeStruct((4 * _EW,), jnp.float32),
        mesh=_mesh(),
        scratch_types=[
            pltpu.VMEM_SHARED((_EW,), jnp.float32),
            pltpu.VMEM_SHARED((_EW,), jnp.float32),
            pltpu.VMEM_SHARED((_EPAD,), jnp.float32),
            pltpu.VMEM_SHARED((_EPAD,), jnp.float32),
            pltpu.VMEM((L, _EB), jnp.float32),
            pltpu.VMEM((L, _EB), jnp.float32),
            pltpu.VMEM((L, _EB), jnp.int32),
            pltpu.VMEM((L, _EB), jnp.int32),
            pltpu.VMEM((_EB,), jnp.int32),
            pltpu.VMEM((_EB,), jnp.int32),
            pltpu.VMEM((_EB,), jnp.int32),
            pltpu.VMEM((_EB,), jnp.int32),
            pltpu.VMEM((_EB,), jnp.float32),
            pltpu.VMEM((2048,), jnp.float32),
            pltpu.VMEM((2048,), jnp.float32),
            pltpu.VMEM((2048,), jnp.float32),
            pltpu.VMEM((_ETILE + L,), jnp.float32),
            pltpu.VMEM((_ETILE + L,), jnp.float32),
            pltpu.SemaphoreType.DMA,
            pltpu.SemaphoreType.DMA,
        ],
    )
    return kfn(eafT, eabT, e0, e1)


def kernel(x, edge_attr, org_x, org_edge_attr, lg_node_idx, edge_index, org_edge_index):
    i0 = lg_node_idx[:, 0].astype(jnp.int32)
    i1 = lg_node_idx[:, 1].astype(jnp.int32)
    e0 = edge_index[0].astype(jnp.int32)
    e1 = edge_index[1].astype(jnp.int32)
    eafT = edge_attr[:, :L].T
    eabT = edge_attr[:, L:].T

    no = _node_call(x, i0, i1)
    graph_x = jnp.concatenate(
        [no[c * _NPAD:c * _NPAD + _NC_OUT] for c in range(2)], axis=0)

    eo = _edge_call(eafT, eabT, e0, e1)
    graph_e = eo.reshape(4, _EPAD, L)[:, :_EC_OUT].reshape(4 * _EC_OUT, L)
    return (graph_x, graph_e)
